# baseline (device time: 65405 ns/iter reference)
import jax
import jax.numpy as jnp
from jax import lax
from jax.experimental import pallas as pl
from jax.experimental.pallas import tpu as pltpu

N_DEV = 16
N_HEADS = 8
DH = 128
SQ = 512
SKV_LOCAL = 2048
D = N_HEADS * DH
SEG = D // N_DEV
BLK = D // 4
SCALE = 0.08838834764831843
LOG2E = 1.4426950408889634
SCALE2 = SCALE * LOG2E


def kernel(x, Wq, Wo, K_ext, V_ext):
    xb = x.reshape(SQ, D).astype(jnp.bfloat16)
    Wqb = Wq.astype(jnp.bfloat16)
    Wob = Wo.astype(jnp.bfloat16)
    Kb = K_ext.reshape(SKV_LOCAL, D).astype(jnp.bfloat16)
    Vb = V_ext.reshape(SKV_LOCAL, D).astype(jnp.bfloat16)

    def body(x_ref, wq_ref, wo_ref, k_ref, v_ref, out_ref,
             catbuf, statbuf, oseg_all, stats_all,
             a_send, a_recv, b_send, b_recv, d_send, d_recv, exit_sems):
        my = lax.axis_index("i")

        barrier = pltpu.get_barrier_semaphore()
        for k in range(1, N_DEV):
            pl.semaphore_signal(barrier, inc=1,
                                device_id=(lax.rem(my + k, N_DEV),),
                                device_id_type=pl.DeviceIdType.MESH)
        pl.semaphore_wait(barrier, N_DEV - 1)

        x2 = x_ref[...]
        for h in range(N_HEADS):
            q = jnp.dot(x2, wq_ref[:, h * DH:(h + 1) * DH],
                        preferred_element_type=jnp.float32)
            q = (q * SCALE2).astype(jnp.bfloat16)
            kh = k_ref[:, h * DH:(h + 1) * DH]
            vh = v_ref[:, h * DH:(h + 1) * DH]
            st = lax.dot_general(kh, q, (((1,), (1,)), ((), ())),
                                 preferred_element_type=jnp.float32)
            m = jnp.max(st, axis=0, keepdims=True)
            p = jnp.exp2(st - m)
            l = jnp.sum(p, axis=0, keepdims=True)
            ot = lax.dot_general(vh, p.astype(jnp.bfloat16),
                                 (((0,), (0,)), ((), ())),
                                 preferred_element_type=jnp.float32)
            catbuf[h * DH:(h + 1) * DH, :] = ot.astype(jnp.bfloat16)
            statbuf[h, 0:1, :] = m
            statbuf[h, 1:2, :] = l
            stat = jnp.concatenate([m, l], axis=0)

            for peer in (2 * h, 2 * h + 1):
                seg = ot[(peer % 2) * SEG:(peer % 2) * SEG + SEG, :]
                seg = seg.astype(jnp.bfloat16)

                @pl.when(peer == my)
                def _(seg=seg, stat=stat):
                    oseg_all[pl.ds(my, 1)] = seg[None]
                    stats_all[pl.ds(my, 1)] = stat[None]

                @pl.when(peer != my)
                def _(h=h, peer=peer):
                    pltpu.make_async_remote_copy(
                        src_ref=statbuf.at[h],
                        dst_ref=stats_all.at[my],
                        send_sem=a_send.at[peer], recv_sem=a_recv.at[my],
                        device_id=(peer,),
                        device_id_type=pl.DeviceIdType.MESH).start()
                    pltpu.make_async_remote_copy(
                        src_ref=catbuf.at[pl.ds(peer * SEG, SEG), :],
                        dst_ref=oseg_all.at[my],
                        send_sem=b_send.at[peer], recv_sem=b_recv.at[my],
                        device_id=(peer,),
                        device_id_type=pl.DeviceIdType.MESH).start()

        for s in range(N_DEV):
            @pl.when(s != my)
            def _(s=s):
                pltpu.make_async_remote_copy(
                    src_ref=stats_all.at[s], dst_ref=stats_all.at[s],
                    send_sem=a_send.at[s], recv_sem=a_recv.at[s],
                    device_id=(my,), device_id_type=pl.DeviceIdType.MESH,
                ).wait_recv()
                pltpu.make_async_remote_copy(
                    src_ref=oseg_all.at[s], dst_ref=oseg_all.at[s],
                    send_sem=b_send.at[s], recv_sem=b_recv.at[s],
                    device_id=(my,), device_id_type=pl.DeviceIdType.MESH,
                ).wait_recv()

        ms = [stats_all[s, 0:1, :] for s in range(N_DEV)]
        ls = [stats_all[s, 1:2, :] for s in range(N_DEV)]
        mg = ms[0]
        for s in range(1, N_DEV):
            mg = jnp.maximum(mg, ms[s])
        num = jnp.zeros((SEG, SQ), jnp.float32)
        den = jnp.zeros((1, SQ), jnp.float32)
        for s in range(N_DEV):
            w = jnp.exp2(ms[s] - mg)
            num = num + oseg_all[s].astype(jnp.float32) * w
            den = den + ls[s] * w
        seg_norm = (num / den).astype(jnp.bfloat16)
        catbuf[pl.ds(my * SEG, SEG), :] = seg_norm

        for peer in range(N_DEV):
            @pl.when(peer != my)
            def _(peer=peer):
                pltpu.make_async_remote_copy(
                    src_ref=catbuf.at[pl.ds(my * SEG, SEG), :],
                    dst_ref=catbuf.at[pl.ds(my * SEG, SEG), :],
                    send_sem=d_send.at[peer], recv_sem=d_recv.at[my],
                    device_id=(peer,),
                    device_id_type=pl.DeviceIdType.MESH).start()

        final = jnp.zeros((SQ, D), jnp.float32)
        for g in range(4):
            for s in range(4 * g, 4 * g + 4):
                @pl.when(s != my)
                def _(s=s):
                    pltpu.make_async_remote_copy(
                        src_ref=catbuf.at[pl.ds(s * SEG, SEG), :],
                        dst_ref=catbuf.at[pl.ds(s * SEG, SEG), :],
                        send_sem=d_send.at[s], recv_sem=d_recv.at[s],
                        device_id=(my,),
                        device_id_type=pl.DeviceIdType.MESH,
                    ).wait_recv()
            final = final + lax.dot_general(
                catbuf[pl.ds(g * BLK, BLK), :],
                wo_ref[pl.ds(g * BLK, BLK), :],
                (((0,), (0,)), ((), ())),
                preferred_element_type=jnp.float32)
        out_ref[...] = final

        for peer in range(N_DEV):
            @pl.when(peer != my)
            def _(peer=peer):
                pltpu.make_async_remote_copy(
                    src_ref=stats_all.at[peer], dst_ref=stats_all.at[peer],
                    send_sem=a_send.at[peer], recv_sem=a_recv.at[peer],
                    device_id=(peer,), device_id_type=pl.DeviceIdType.MESH,
                ).wait_send()
                pltpu.make_async_remote_copy(
                    src_ref=oseg_all.at[peer], dst_ref=oseg_all.at[peer],
                    send_sem=b_send.at[peer], recv_sem=b_recv.at[peer],
                    device_id=(peer,), device_id_type=pl.DeviceIdType.MESH,
                ).wait_send()
                pltpu.make_async_remote_copy(
                    src_ref=catbuf.at[pl.ds(peer * SEG, SEG), :],
                    dst_ref=catbuf.at[pl.ds(peer * SEG, SEG), :],
                    send_sem=d_send.at[peer], recv_sem=d_recv.at[peer],
                    device_id=(peer,), device_id_type=pl.DeviceIdType.MESH,
                ).wait_send()

        for k in range(1, N_DEV):
            pl.semaphore_signal(exit_sems.at[N_DEV - k - 1], inc=1,
                                device_id=(lax.rem(my + k, N_DEV),),
                                device_id_type=pl.DeviceIdType.MESH)
        for j in range(1, N_DEV):
            pl.semaphore_wait(exit_sems.at[j - 1], 1)

    out = pl.pallas_call(
        body,
        out_shape=jax.ShapeDtypeStruct((SQ, D), jnp.float32),
        in_specs=[pl.BlockSpec(memory_space=pltpu.VMEM)] * 5,
        out_specs=pl.BlockSpec(memory_space=pltpu.VMEM),
        scratch_shapes=[
            pltpu.VMEM((D, SQ), jnp.bfloat16),
            pltpu.VMEM((N_HEADS, 2, SQ), jnp.float32),
            pltpu.VMEM((N_DEV, SEG, SQ), jnp.bfloat16),
            pltpu.VMEM((N_DEV, 2, SQ), jnp.float32),
            pltpu.SemaphoreType.DMA((N_DEV,)),
            pltpu.SemaphoreType.DMA((N_DEV,)),
            pltpu.SemaphoreType.DMA((N_DEV,)),
            pltpu.SemaphoreType.DMA((N_DEV,)),
            pltpu.SemaphoreType.DMA((N_DEV,)),
            pltpu.SemaphoreType.DMA((N_DEV,)),
            pltpu.SemaphoreType.REGULAR((N_DEV - 1,)),
        ],
        compiler_params=pltpu.CompilerParams(collective_id=0),
    )(xb, Wqb, Wob, Kb, Vb)
    return out.reshape(1, SQ, D)
